# baseline (speedup 1.0000x reference)
import functools

import jax
import jax.numpy as jnp
from jax import lax
from jax.experimental import pallas as pl
from jax.experimental.pallas import tpu as pltpu
from jax.experimental.pallas import tpu_sc as plsc

VOCAB = 10000
EMB = 64
SEQ = 1000
BATCH = 1024
H1 = 16
H2 = 5

NC = 2
NS = 16
NW = NC * NS
BPW = BATCH // NW
CHUNK = 200
NCH = SEQ // CHUNK
KTOT = BPW * NCH
LANES = 16
EGRP = EMB // LANES

_mesh = plsc.VectorSubcoreMesh(
    core_axis_name="c", subcore_axis_name="s", num_cores=NC, num_subcores=NS
)


@functools.partial(
    pl.kernel,
    out_type=jax.ShapeDtypeStruct((BATCH * EMB,), jnp.float32),
    mesh=_mesh,
    compiler_params=pltpu.CompilerParams(use_tc_tiling_on_sc=False),
    scratch_types=[
        pltpu.VMEM((BPW * SEQ,), jnp.int32),
        pltpu.VMEM((CHUNK, EMB), jnp.float32),
        pltpu.VMEM((CHUNK, EMB), jnp.float32),
        pltpu.VMEM((CHUNK, EMB), jnp.float32),
        pltpu.VMEM((CHUNK, EMB), jnp.float32),
        pltpu.VMEM((CHUNK, EMB), jnp.float32),
        pltpu.VMEM((CHUNK, EMB), jnp.float32),
        pltpu.VMEM((BPW * EMB,), jnp.float32),
        pltpu.SemaphoreType.DMA,
        pltpu.SemaphoreType.DMA,
        pltpu.SemaphoreType.DMA,
        pltpu.SemaphoreType.DMA,
        pltpu.SemaphoreType.DMA,
        pltpu.SemaphoreType.DMA,
    ],
)
def _pooled_sum(inputs_hbm, table_hbm, out_hbm, idx_v, rows0, rows1, rows2,
                rows3, rows4, rows5, pooled_v, sem0, sem1, sem2, sem3, sem4,
                sem5):
    wid = lax.axis_index("s") * NC + lax.axis_index("c")
    base = pl.multiple_of(wid * (BPW * SEQ), 8)
    pltpu.sync_copy(inputs_hbm.at[pl.ds(base, BPW * SEQ)], idx_v)

    def zero_body(i, carry):
        off = pl.multiple_of(i * LANES, 8)
        pooled_v[pl.ds(off, LANES)] = jnp.zeros((LANES,), jnp.float32)
        return carry

    lax.fori_loop(0, BPW * EMB // LANES, zero_body, 0)

    def issue(k, rows, sem):
        off = pl.multiple_of(k * CHUNK, 8)
        return pltpu.async_copy(
            table_hbm.at[idx_v.at[pl.ds(off, CHUNK)]], rows, sem
        )

    def wait(rows, sem):
        pltpu.make_async_copy(
            table_hbm.at[idx_v.at[pl.ds(0, CHUNK)]], rows, sem
        ).wait()

    def accum_chunk(rows, k):
        row = k // NCH
        off = pl.multiple_of(row * EMB, 8)
        plsc.addupdate(pooled_v.at[pl.ds(off, LANES)], rows[0, pl.ds(0, LANES)])

    bufs = (rows0, rows1, rows2, rows3, rows4, rows5)
    sems = (sem0, sem1, sem2, sem3, sem4, sem5)
    NB = 5

    for b in range(NB):
        issue(b, bufs[b], sems[b])

    def body2(t, carry):
        k0 = NB * t
        for b in range(NB):
            wait(bufs[b], sems[b])
            accum_chunk(bufs[b], k0 + b)
            issue(k0 + b + NB, bufs[b], sems[b])
        return carry

    lax.fori_loop(0, (KTOT - NB) // NB, body2, 0)
    for b in range(NB):
        wait(bufs[b], sems[b])
        accum_chunk(bufs[b], KTOT - NB + b)

    obase = pl.multiple_of(wid * (BPW * EMB), 8)
    pltpu.sync_copy(pooled_v, out_hbm.at[pl.ds(obase, BPW * EMB)])


def _mlp_body(pooled_ref, w1_ref, b1_ref, w2_ref, b2_ref, out_ref):
    pooled = pooled_ref[...] * (1.0 / SEQ)
    h = jnp.dot(pooled, w1_ref[...], preferred_element_type=jnp.float32)
    h = jnp.maximum(h + b1_ref[...], 0.0)
    z = jnp.dot(h, w2_ref[...], preferred_element_type=jnp.float32) + b2_ref[...]
    out_ref[...] = 1.0 / (1.0 + jnp.exp(-z))


def kernel(inputs, table, W1, b1, W2, b2):
    pooled_sum = _pooled_sum(inputs.reshape(BATCH * SEQ), table)
    return pl.pallas_call(
        _mlp_body,
        out_shape=jax.ShapeDtypeStruct((BATCH, H2), jnp.float32),
    )(pooled_sum.reshape(BATCH, EMB), W1, b1.reshape(1, H1), W2, b2.reshape(1, H2))


# speedup vs baseline: 1.0151x; 1.0151x over previous
import functools

import jax
import jax.numpy as jnp
from jax import lax
from jax.experimental import pallas as pl
from jax.experimental.pallas import tpu as pltpu
from jax.experimental.pallas import tpu_sc as plsc

VOCAB = 10000
EMB = 64
SEQ = 1000
BATCH = 1024
H1 = 16
H2 = 5

NC = 2
NS = 16
NW = NC * NS
BPW = BATCH // NW
CHUNK = 200
NCH = SEQ // CHUNK
KTOT = BPW * NCH
LANES = 16
EGRP = EMB // LANES

_mesh = plsc.VectorSubcoreMesh(
    core_axis_name="c", subcore_axis_name="s", num_cores=NC, num_subcores=NS
)


@functools.partial(
    pl.kernel,
    out_type=jax.ShapeDtypeStruct((BATCH * EMB,), jnp.float32),
    mesh=_mesh,
    compiler_params=pltpu.CompilerParams(use_tc_tiling_on_sc=False),
    scratch_types=[
        pltpu.VMEM((BPW * SEQ,), jnp.int32),
        pltpu.VMEM((CHUNK, EMB), jnp.float32),
        pltpu.VMEM((CHUNK, EMB), jnp.float32),
        pltpu.VMEM((CHUNK, EMB), jnp.float32),
        pltpu.VMEM((CHUNK, EMB), jnp.float32),
        pltpu.VMEM((BPW * EMB,), jnp.float32),
        pltpu.SemaphoreType.DMA,
        pltpu.SemaphoreType.DMA,
        pltpu.SemaphoreType.DMA,
        pltpu.SemaphoreType.DMA,
    ],
)
def _pooled_sum(inputs_hbm, table_hbm, out_hbm, idx_v, rows0, rows1, rows2,
                rows3, pooled_v, sem0, sem1, sem2, sem3):
    wid = lax.axis_index("s") * NC + lax.axis_index("c")
    base = pl.multiple_of(wid * (BPW * SEQ), 8)
    pltpu.sync_copy(inputs_hbm.at[pl.ds(base, BPW * SEQ)], idx_v)

    def zero_body(i, carry):
        off = pl.multiple_of(i * LANES, 8)
        pooled_v[pl.ds(off, LANES)] = jnp.zeros((LANES,), jnp.float32)
        return carry

    lax.fori_loop(0, BPW * EMB // LANES, zero_body, 0)

    def issue(k, rows, sem):
        off = pl.multiple_of(k * CHUNK, 8)
        return pltpu.async_copy(
            table_hbm.at[idx_v.at[pl.ds(off, CHUNK)]], rows, sem
        )

    def wait(rows, sem):
        pltpu.make_async_copy(
            table_hbm.at[idx_v.at[pl.ds(0, CHUNK)]], rows, sem
        ).wait()

    def accum_chunk(rows, k):
        def body(i, accs):
            return tuple(
                accs[j] + rows[i, pl.ds(LANES * j, LANES)] for j in range(EGRP)
            )

        accs = lax.fori_loop(
            0, CHUNK, body,
            tuple(jnp.zeros((LANES,), jnp.float32) for _ in range(EGRP)),
            unroll=4,
        )
        row = k // NCH
        for j in range(EGRP):
            off = pl.multiple_of(row * EMB + LANES * j, 8)
            plsc.addupdate(pooled_v.at[pl.ds(off, LANES)], accs[j])

    bufs = (rows0, rows1, rows2, rows3)
    sems = (sem0, sem1, sem2, sem3)
    NB = 4

    for b in range(NB):
        issue(b, bufs[b], sems[b])

    def body2(t, carry):
        k0 = NB * t
        for b in range(NB):
            wait(bufs[b], sems[b])
            accum_chunk(bufs[b], k0 + b)
            issue(k0 + b + NB, bufs[b], sems[b])
        return carry

    lax.fori_loop(0, (KTOT - NB) // NB, body2, 0)
    for b in range(NB):
        wait(bufs[b], sems[b])
        accum_chunk(bufs[b], KTOT - NB + b)

    obase = pl.multiple_of(wid * (BPW * EMB), 8)
    pltpu.sync_copy(pooled_v, out_hbm.at[pl.ds(obase, BPW * EMB)])


def _mlp_body(pooled_ref, w1_ref, b1_ref, w2_ref, b2_ref, out_ref):
    pooled = pooled_ref[...] * (1.0 / SEQ)
    h = jnp.dot(pooled, w1_ref[...], preferred_element_type=jnp.float32)
    h = jnp.maximum(h + b1_ref[...], 0.0)
    z = jnp.dot(h, w2_ref[...], preferred_element_type=jnp.float32) + b2_ref[...]
    out_ref[...] = 1.0 / (1.0 + jnp.exp(-z))


def kernel(inputs, table, W1, b1, W2, b2):
    pooled_sum = _pooled_sum(inputs.reshape(BATCH * SEQ), table)
    return pl.pallas_call(
        _mlp_body,
        out_shape=jax.ShapeDtypeStruct((BATCH, H2), jnp.float32),
    )(pooled_sum.reshape(BATCH, EMB), W1, b1.reshape(1, H1), W2, b2.reshape(1, H2))
